# baseline (speedup 1.0000x reference)
import functools

import jax
import jax.numpy as jnp
from jax import lax
from jax.experimental import pallas as pl
from jax.experimental.pallas import tpu as pltpu
from jax.experimental.pallas import tpu_sc as plsc

T, H, E, K, F, FS = 2048, 1024, 8, 2, 2048, 2048
AUX_COEFF = 0.01
EP = 128
RTB = 256
XTB = 512
BLK = 256
NROWS = T * K + E * BLK
NBLK = NROWS // BLK
NBLK_PAD = 32
NW = 32
TPW = T // NW

_MI1, _MI2, _MR1, _MR2, _MW1, _MW2 = 0, 1, 2, 3, 4, 5


def _router_body(x_ref, rw_ref, meta_ref, cnt_ref, aux_ref, scnt, sprob):
    i = pl.program_id(0)
    x = x_ref[...]
    logits = lax.dot_general(x, rw_ref[...], (((1,), (0,)), ((), ())),
                             preferred_element_type=jnp.float32)
    lane = lax.broadcasted_iota(jnp.int32, (RTB, EP), 1)
    logits = jnp.where(lane < E, logits, -1e30)
    m = jnp.max(logits, axis=-1, keepdims=True)
    ex = jnp.exp(logits - m)
    probs = ex / jnp.sum(ex, axis=-1, keepdims=True)
    m1 = jnp.max(probs, axis=-1, keepdims=True)
    i1 = jnp.min(jnp.where(probs == m1, lane, EP), axis=-1, keepdims=True)
    mask1 = lane == i1
    p2 = jnp.where(mask1, -1.0, probs)
    m2 = jnp.max(p2, axis=-1, keepdims=True)
    i2 = jnp.min(jnp.where(p2 == m2, lane, EP), axis=-1, keepdims=True)
    mask2 = lane == i2
    routing = jnp.where(mask1 | mask2, 1.0, 0.0)

    @pl.when(i == 0)
    def _():
        scnt[...] = jnp.zeros_like(scnt)
        sprob[...] = jnp.zeros_like(sprob)

    r_iota = lax.broadcasted_iota(jnp.int32, (RTB, RTB), 0)
    c_iota = lax.broadcasted_iota(jnp.int32, (RTB, RTB), 1)
    lstrict = (r_iota > c_iota).astype(jnp.float32)
    rank_all = scnt[...] + lax.dot_general(
        lstrict, routing, (((1,), (0,)), ((), ())),
        preferred_element_type=jnp.float32)
    rank1 = jnp.sum(jnp.where(mask1, rank_all, 0.0), axis=-1, keepdims=True)
    rank2 = jnp.sum(jnp.where(mask2, rank_all, 0.0), axis=-1, keepdims=True)

    def oh(j):
        return (lane == j).astype(jnp.float32)

    meta = (i1.astype(jnp.float32) * oh(_MI1) + i2.astype(jnp.float32) * oh(_MI2)
            + rank1 * oh(_MR1) + rank2 * oh(_MR2) + m1 * oh(_MW1) + m2 * oh(_MW2))
    ident = (r_iota == c_iota).astype(jnp.float32)
    meta_ref[...] = lax.dot_general(meta, ident, (((0,), (0,)), ((), ())),
                                    preferred_element_type=jnp.float32,
                                    precision=lax.Precision.HIGHEST)

    scnt[...] += jnp.sum(routing, axis=0, keepdims=True)
    sprob[...] += jnp.sum(probs, axis=0, keepdims=True)

    @pl.when(i == pl.num_programs(0) - 1)
    def _():
        frac = scnt[...] / (T * K)
        pmean = sprob[...] / T
        cnt_ref[...] = scnt[...]
        aux_ref[...] = jnp.reshape(AUX_COEFF * E * jnp.sum(frac * pmean),
                                   (1, 1))


_BLK_SH = BLK.bit_length() - 1


def _sc_offsets(cntbuf):
    cnt = cntbuf[...].astype(jnp.int32)
    padded = ((cnt + (BLK - 1)) >> _BLK_SH) << _BLK_SH
    lanes = lax.iota(jnp.int32, 16)
    x = padded
    for sh in (1, 2, 4, 8):
        prev = x.at[jnp.maximum(lanes - sh, 0)].get(mode="promise_in_bounds")
        x = x + jnp.where(lanes >= sh, prev, 0)
    off = x - padded
    return cnt, padded, off


def _dispatch_body(x_hbm, meta_hbm, counts_hbm, xs_hbm, cnt_hbm, bexp_hbm,
                   xbuf, metabuf, cntbuf, idxbuf, cntob, bexpb, sem):
    wid = lax.axis_index("s") * 2 + lax.axis_index("c")
    base = wid * TPW
    hs = [pltpu.async_copy(counts_hbm.at[0, pl.ds(0, 16)], cntbuf, sem)]
    for r in (_MI1, _MI2, _MR1, _MR2):
        hs.append(pltpu.async_copy(meta_hbm.at[r, pl.ds(base, TPW)],
                                   metabuf.at[r], sem))
    hs.append(pltpu.async_copy(x_hbm.at[pl.ds(base, TPW)], xbuf, sem))
    for hh in hs:
        hh.wait()
    cnt, padded, off = _sc_offsets(cntbuf)
    for c in range(TPW // 16):
        sl = pl.ds(c * 16, 16)
        e1 = metabuf[_MI1, sl].astype(jnp.int32)
        e2 = metabuf[_MI2, sl].astype(jnp.int32)
        r1 = metabuf[_MR1, sl].astype(jnp.int32)
        r2 = metabuf[_MR2, sl].astype(jnp.int32)
        idxbuf[0, sl] = off.at[e1].get(mode="promise_in_bounds") + r1
        idxbuf[1, sl] = off.at[e2].get(mode="promise_in_bounds") + r2
    s1 = pltpu.async_copy(xbuf, xs_hbm.at[idxbuf.at[0]], sem)
    s2 = pltpu.async_copy(xbuf, xs_hbm.at[idxbuf.at[1]], sem)

    @pl.when(wid == 0)
    def _():
        lanes = lax.iota(jnp.int32, 16)
        nblk = padded >> _BLK_SH
        blkbase = off >> _BLK_SH
        ends = blkbase + nblk
        sev = jnp.full((16,), 7, dtype=jnp.int32)
        totblk = (off + padded).at[sev].get(mode="promise_in_bounds") >> _BLK_SH
        cntob[...] = jnp.where(lanes == 8, totblk, cnt)
        pltpu.sync_copy(cntob, cnt_hbm)
        for c in range(NBLK_PAD // 16):
            sv = lanes + c * 16
            acc = jnp.zeros((16,), dtype=jnp.int32)
            for e in range(E):
                esplat = jnp.full((16,), e, dtype=jnp.int32)
                end_e = ends.at[esplat].get(mode="promise_in_bounds")
                acc = acc + jnp.where(sv >= end_e, 1, 0)
            bexpb[pl.ds(c * 16, 16)] = jnp.minimum(acc, E - 1)
        pltpu.sync_copy(bexpb, bexp_hbm)

    s1.wait()
    s2.wait()


def _shared_body(x_ref, sw1_ref, sw2_ref, moe_ref, out_ref):
    h = jax.nn.gelu(lax.dot_general(
        x_ref[...], sw1_ref[...], (((1,), (0,)), ((), ())),
        preferred_element_type=jnp.float32))
    out_ref[...] = moe_ref[...] + lax.dot_general(
        h, sw2_ref[...], (((1,), (0,)), ((), ())),
        preferred_element_type=jnp.float32)


def _group_mlp_body(cnt_ref, bexp_ref, xs_ref, w1_ref, w2_ref, out_ref):
    s = pl.program_id(0)

    @pl.when(s < cnt_ref[8])
    def _():
        h = jax.nn.gelu(lax.dot_general(
            xs_ref[...], w1_ref[0], (((1,), (0,)), ((), ())),
            preferred_element_type=jnp.float32))
        out_ref[...] = lax.dot_general(h, w2_ref[0], (((1,), (0,)), ((), ())),
                                       preferred_element_type=jnp.float32)


def _combine_body(ys_hbm, meta_hbm, counts_hbm, out_hbm,
                  ya0, yb0, ya1, yb1, obuf,
                  metabuf, cntbuf, idxbuf, sem0, sem1, osem):
    wid = lax.axis_index("s") * 2 + lax.axis_index("c")
    base = wid * TPW
    pltpu.sync_copy(counts_hbm.at[0, pl.ds(0, 16)], cntbuf)
    _, _, off = _sc_offsets(cntbuf)
    for r in (_MI1, _MI2, _MR1, _MR2, _MW1, _MW2):
        pltpu.sync_copy(meta_hbm.at[r, pl.ds(base, TPW)], metabuf.at[r])
    bufs = ((ya0, yb0, sem0), (ya1, yb1, sem1))
    nch = TPW // 16

    def start(c):
        ya, yb, sem = bufs[c % 2]
        sl = pl.ds(c * 16, 16)
        e1 = metabuf[_MI1, sl].astype(jnp.int32)
        e2 = metabuf[_MI2, sl].astype(jnp.int32)
        r1 = metabuf[_MR1, sl].astype(jnp.int32)
        r2 = metabuf[_MR2, sl].astype(jnp.int32)
        ir = 2 * (c % 2)
        idxbuf[ir, pl.ds(0, 16)] = off.at[e1].get(mode="promise_in_bounds") + r1
        idxbuf[ir + 1, pl.ds(0, 16)] = (off.at[e2].get(mode="promise_in_bounds")
                                        + r2)
        ha = pltpu.async_copy(ys_hbm.at[idxbuf.at[ir]], ya, sem)
        hb = pltpu.async_copy(ys_hbm.at[idxbuf.at[ir + 1]], yb, sem)
        return (ha, hb)

    handles = {0: start(0), 1: start(1)}
    oh = {}
    for c in range(nch):
        for hh in handles.pop(c):
            hh.wait()
        if c > 0:
            oh[c - 1].wait()
        ya, yb, _ = bufs[c % 2]
        wa16 = metabuf[_MW1, pl.ds(c * 16, 16)]
        wb16 = metabuf[_MW2, pl.ds(c * 16, 16)]

        def jbody(j, _, ya=ya, yb=yb, wa16=wa16, wb16=wb16):
            splat = jnp.full((16,), j, dtype=jnp.int32)
            wa = wa16.at[splat].get(mode="promise_in_bounds")
            wb = wb16.at[splat].get(mode="promise_in_bounds")
            for cc in range(H // 16):
                csl = pl.ds(cc * 16, 16)
                obuf[j, csl] = wa * ya[j, csl] + wb * yb[j, csl]
            return 0

        lax.fori_loop(0, 16, jbody, 0)
        oh[c] = pltpu.async_copy(obuf, out_hbm.at[pl.ds(base + c * 16, 16)],
                                 osem)
        if c + 2 < nch:
            handles[c + 2] = start(c + 2)
    oh[nch - 1].wait()


def _sc_mesh():
    return plsc.VectorSubcoreMesh(core_axis_name="c", subcore_axis_name="s")


@jax.jit
def kernel(hidden_states, router_w, w1, w2, shared_w1, shared_w2):
    rwp = jnp.pad(router_w, ((0, 0), (0, EP - E)))

    meta_t, counts, aux = pl.pallas_call(
        _router_body,
        grid=(T // RTB,),
        in_specs=[
            pl.BlockSpec((RTB, H), lambda i: (i, 0)),
            pl.BlockSpec((H, EP), lambda i: (0, 0)),
        ],
        out_specs=[
            pl.BlockSpec((EP, RTB), lambda i: (0, i)),
            pl.BlockSpec((1, EP), lambda i: (0, 0)),
            pl.BlockSpec((1, 1), lambda i: (0, 0)),
        ],
        out_shape=[
            jax.ShapeDtypeStruct((EP, T), jnp.float32),
            jax.ShapeDtypeStruct((1, EP), jnp.float32),
            jax.ShapeDtypeStruct((1, 1), jnp.float32),
        ],
        scratch_shapes=[
            pltpu.VMEM((1, EP), jnp.float32),
            pltpu.VMEM((1, EP), jnp.float32),
        ],
    )(hidden_states, rwp)

    dispatch = functools.partial(
        pl.kernel,
        mesh=_sc_mesh(),
        out_type=[
            jax.ShapeDtypeStruct((NROWS, H), jnp.float32),
            jax.ShapeDtypeStruct((16,), jnp.int32),
            jax.ShapeDtypeStruct((NBLK_PAD,), jnp.int32),
        ],
        scratch_types=[
            pltpu.VMEM((TPW, H), jnp.float32),
            pltpu.VMEM((8, TPW), jnp.float32),
            pltpu.VMEM((16,), jnp.float32),
            pltpu.VMEM((2, TPW), jnp.int32),
            pltpu.VMEM((16,), jnp.int32),
            pltpu.VMEM((NBLK_PAD,), jnp.int32),
            pltpu.SemaphoreType.DMA,
        ],
    )
    xs, cnt16, bexp = dispatch(_dispatch_body)(hidden_states, meta_t, counts)

    ys = pl.pallas_call(
        _group_mlp_body,
        grid_spec=pltpu.PrefetchScalarGridSpec(
            num_scalar_prefetch=2,
            grid=(NBLK,),
            in_specs=[
                pl.BlockSpec((BLK, H), lambda s, cnt, bexp: (s, 0)),
                pl.BlockSpec((1, H, F), lambda s, cnt, bexp: (bexp[s], 0, 0)),
                pl.BlockSpec((1, F, H), lambda s, cnt, bexp: (bexp[s], 0, 0)),
            ],
            out_specs=pl.BlockSpec((BLK, H), lambda s, cnt, bexp: (s, 0)),
        ),
        out_shape=jax.ShapeDtypeStruct((NROWS, H), jnp.float32),
        compiler_params=pltpu.CompilerParams(
            dimension_semantics=("arbitrary",)),
    )(cnt16, bexp, xs, w1, w2)

    combine = functools.partial(
        pl.kernel,
        mesh=_sc_mesh(),
        out_type=jax.ShapeDtypeStruct((T, H), jnp.float32),
        scratch_types=[
            pltpu.VMEM((16, H), jnp.float32),
            pltpu.VMEM((16, H), jnp.float32),
            pltpu.VMEM((16, H), jnp.float32),
            pltpu.VMEM((16, H), jnp.float32),
            pltpu.VMEM((16, H), jnp.float32),
            pltpu.VMEM((8, TPW), jnp.float32),
            pltpu.VMEM((16,), jnp.float32),
            pltpu.VMEM((4, 16), jnp.int32),
            pltpu.SemaphoreType.DMA,
            pltpu.SemaphoreType.DMA,
            pltpu.SemaphoreType.DMA,
        ],
    )
    moe = combine(_combine_body)(ys, meta_t, counts)

    combined = pl.pallas_call(
        _shared_body,
        grid=(T // XTB,),
        in_specs=[
            pl.BlockSpec((XTB, H), lambda t: (t, 0)),
            pl.BlockSpec((H, FS), lambda t: (0, 0)),
            pl.BlockSpec((FS, H), lambda t: (0, 0)),
            pl.BlockSpec((XTB, H), lambda t: (t, 0)),
        ],
        out_specs=pl.BlockSpec((XTB, H), lambda t: (t, 0)),
        out_shape=jax.ShapeDtypeStruct((T, H), jnp.float32),
    )(hidden_states, shared_w1, shared_w2, moe)

    return combined, aux[0, 0]


# speedup vs baseline: 1.0225x; 1.0225x over previous
import functools

import jax
import jax.numpy as jnp
from jax import lax
from jax.experimental import pallas as pl
from jax.experimental.pallas import tpu as pltpu
from jax.experimental.pallas import tpu_sc as plsc

T, H, E, K, F, FS = 2048, 1024, 8, 2, 2048, 2048
AUX_COEFF = 0.01
EP = 128
RTB = 512
XTB = 512
BLK = 256
NROWS = T * K + E * BLK
NBLK = NROWS // BLK
NBLK_PAD = 32
NW = 32
TPW = T // NW

_MI1, _MI2, _MR1, _MR2, _MW1, _MW2 = 0, 1, 2, 3, 4, 5


def _router_body(x_ref, rw_ref, meta_ref, cnt_ref, aux_ref, scnt, sprob):
    i = pl.program_id(0)
    x = x_ref[...]
    logits = lax.dot_general(x, rw_ref[...], (((1,), (0,)), ((), ())),
                             preferred_element_type=jnp.float32)
    lane = lax.broadcasted_iota(jnp.int32, (RTB, EP), 1)
    logits = jnp.where(lane < E, logits, -1e30)
    m = jnp.max(logits, axis=-1, keepdims=True)
    ex = jnp.exp(logits - m)
    probs = ex / jnp.sum(ex, axis=-1, keepdims=True)
    m1 = jnp.max(probs, axis=-1, keepdims=True)
    i1 = jnp.min(jnp.where(probs == m1, lane, EP), axis=-1, keepdims=True)
    mask1 = lane == i1
    p2 = jnp.where(mask1, -1.0, probs)
    m2 = jnp.max(p2, axis=-1, keepdims=True)
    i2 = jnp.min(jnp.where(p2 == m2, lane, EP), axis=-1, keepdims=True)
    mask2 = lane == i2
    routing = jnp.where(mask1 | mask2, 1.0, 0.0)

    @pl.when(i == 0)
    def _():
        scnt[...] = jnp.zeros_like(scnt)
        sprob[...] = jnp.zeros_like(sprob)

    r_iota = lax.broadcasted_iota(jnp.int32, (RTB, RTB), 0)
    c_iota = lax.broadcasted_iota(jnp.int32, (RTB, RTB), 1)
    lstrict = (r_iota > c_iota).astype(jnp.float32)
    rank_all = scnt[...] + lax.dot_general(
        lstrict, routing, (((1,), (0,)), ((), ())),
        preferred_element_type=jnp.float32)
    rank1 = jnp.sum(jnp.where(mask1, rank_all, 0.0), axis=-1, keepdims=True)
    rank2 = jnp.sum(jnp.where(mask2, rank_all, 0.0), axis=-1, keepdims=True)

    def oh(j):
        return (lane == j).astype(jnp.float32)

    meta = (i1.astype(jnp.float32) * oh(_MI1) + i2.astype(jnp.float32) * oh(_MI2)
            + rank1 * oh(_MR1) + rank2 * oh(_MR2) + m1 * oh(_MW1) + m2 * oh(_MW2))
    ident = (r_iota == c_iota).astype(jnp.float32)
    meta_ref[...] = lax.dot_general(meta, ident, (((0,), (0,)), ((), ())),
                                    preferred_element_type=jnp.float32,
                                    precision=lax.Precision.HIGHEST)

    scnt[...] += jnp.sum(routing, axis=0, keepdims=True)
    sprob[...] += jnp.sum(probs, axis=0, keepdims=True)

    @pl.when(i == pl.num_programs(0) - 1)
    def _():
        frac = scnt[...] / (T * K)
        pmean = sprob[...] / T
        cnt_ref[...] = scnt[...]
        aux_ref[...] = jnp.reshape(AUX_COEFF * E * jnp.sum(frac * pmean),
                                   (1, 1))


_BLK_SH = BLK.bit_length() - 1


def _sc_offsets(cntbuf):
    cnt = cntbuf[...].astype(jnp.int32)
    padded = ((cnt + (BLK - 1)) >> _BLK_SH) << _BLK_SH
    lanes = lax.iota(jnp.int32, 16)
    x = padded
    for sh in (1, 2, 4, 8):
        prev = x.at[jnp.maximum(lanes - sh, 0)].get(mode="promise_in_bounds")
        x = x + jnp.where(lanes >= sh, prev, 0)
    off = x - padded
    return cnt, padded, off


def _dispatch_body(x_hbm, meta_hbm, counts_hbm, xs_hbm, cnt_hbm, bexp_hbm,
                   xbuf, metabuf, cntbuf, idxbuf, cntob, bexpb, sem):
    wid = lax.axis_index("s") * 2 + lax.axis_index("c")
    base = wid * TPW
    hs = [pltpu.async_copy(counts_hbm.at[0, pl.ds(0, 16)], cntbuf, sem)]
    for r in (_MI1, _MI2, _MR1, _MR2):
        hs.append(pltpu.async_copy(meta_hbm.at[r, pl.ds(base, TPW)],
                                   metabuf.at[r], sem))
    hs.append(pltpu.async_copy(x_hbm.at[pl.ds(base, TPW)], xbuf, sem))
    for hh in hs:
        hh.wait()
    cnt, padded, off = _sc_offsets(cntbuf)
    for c in range(TPW // 16):
        sl = pl.ds(c * 16, 16)
        e1 = metabuf[_MI1, sl].astype(jnp.int32)
        e2 = metabuf[_MI2, sl].astype(jnp.int32)
        r1 = metabuf[_MR1, sl].astype(jnp.int32)
        r2 = metabuf[_MR2, sl].astype(jnp.int32)
        idxbuf[0, sl] = off.at[e1].get(mode="promise_in_bounds") + r1
        idxbuf[1, sl] = off.at[e2].get(mode="promise_in_bounds") + r2
    s1 = pltpu.async_copy(xbuf, xs_hbm.at[idxbuf.at[0]], sem)
    s2 = pltpu.async_copy(xbuf, xs_hbm.at[idxbuf.at[1]], sem)

    @pl.when(wid == 0)
    def _():
        lanes = lax.iota(jnp.int32, 16)
        nblk = padded >> _BLK_SH
        blkbase = off >> _BLK_SH
        ends = blkbase + nblk
        sev = jnp.full((16,), 7, dtype=jnp.int32)
        totblk = (off + padded).at[sev].get(mode="promise_in_bounds") >> _BLK_SH
        cntob[...] = jnp.where(lanes == 8, totblk, cnt)
        pltpu.sync_copy(cntob, cnt_hbm)
        for c in range(NBLK_PAD // 16):
            sv = lanes + c * 16
            acc = jnp.zeros((16,), dtype=jnp.int32)
            for e in range(E):
                esplat = jnp.full((16,), e, dtype=jnp.int32)
                end_e = ends.at[esplat].get(mode="promise_in_bounds")
                acc = acc + jnp.where(sv >= end_e, 1, 0)
            bexpb[pl.ds(c * 16, 16)] = jnp.minimum(acc, E - 1)
        pltpu.sync_copy(bexpb, bexp_hbm)

    s1.wait()
    s2.wait()


def _shared_body(x_ref, sw1_ref, sw2_ref, moe_ref, out_ref):
    h = jax.nn.gelu(lax.dot_general(
        x_ref[...], sw1_ref[...], (((1,), (0,)), ((), ())),
        preferred_element_type=jnp.float32))
    out_ref[...] = moe_ref[...] + lax.dot_general(
        h, sw2_ref[...], (((1,), (0,)), ((), ())),
        preferred_element_type=jnp.float32)


def _group_mlp_body(cnt_ref, bexp_ref, xs_ref, w1_ref, w2_ref, out_ref):
    s = pl.program_id(0)

    @pl.when(s < cnt_ref[8])
    def _():
        h = jax.nn.gelu(lax.dot_general(
            xs_ref[...], w1_ref[0], (((1,), (0,)), ((), ())),
            preferred_element_type=jnp.float32))
        out_ref[...] = lax.dot_general(h, w2_ref[0], (((1,), (0,)), ((), ())),
                                       preferred_element_type=jnp.float32)


def _combine_body(ys_hbm, meta_hbm, counts_hbm, out_hbm,
                  ya0, yb0, ya1, yb1, obuf,
                  metabuf, cntbuf, idxbuf, sem0, sem1, osem):
    wid = lax.axis_index("s") * 2 + lax.axis_index("c")
    base = wid * TPW
    pltpu.sync_copy(counts_hbm.at[0, pl.ds(0, 16)], cntbuf)
    _, _, off = _sc_offsets(cntbuf)
    for r in (_MI1, _MI2, _MR1, _MR2, _MW1, _MW2):
        pltpu.sync_copy(meta_hbm.at[r, pl.ds(base, TPW)], metabuf.at[r])
    bufs = ((ya0, yb0, sem0), (ya1, yb1, sem1))
    nch = TPW // 16

    def start(c):
        ya, yb, sem = bufs[c % 2]
        sl = pl.ds(c * 16, 16)
        e1 = metabuf[_MI1, sl].astype(jnp.int32)
        e2 = metabuf[_MI2, sl].astype(jnp.int32)
        r1 = metabuf[_MR1, sl].astype(jnp.int32)
        r2 = metabuf[_MR2, sl].astype(jnp.int32)
        ir = 2 * (c % 2)
        idxbuf[ir, pl.ds(0, 16)] = off.at[e1].get(mode="promise_in_bounds") + r1
        idxbuf[ir + 1, pl.ds(0, 16)] = (off.at[e2].get(mode="promise_in_bounds")
                                        + r2)
        ha = pltpu.async_copy(ys_hbm.at[idxbuf.at[ir]], ya, sem)
        hb = pltpu.async_copy(ys_hbm.at[idxbuf.at[ir + 1]], yb, sem)
        return (ha, hb)

    handles = {0: start(0), 1: start(1)}
    oh = {}
    for c in range(nch):
        for hh in handles.pop(c):
            hh.wait()
        if c > 0:
            oh[c - 1].wait()
        ya, yb, _ = bufs[c % 2]
        wa16 = metabuf[_MW1, pl.ds(c * 16, 16)]
        wb16 = metabuf[_MW2, pl.ds(c * 16, 16)]

        def jbody(j, _, ya=ya, yb=yb, wa16=wa16, wb16=wb16):
            splat = jnp.full((16,), j, dtype=jnp.int32)
            wa = wa16.at[splat].get(mode="promise_in_bounds")
            wb = wb16.at[splat].get(mode="promise_in_bounds")
            for cc in range(H // 16):
                csl = pl.ds(cc * 16, 16)
                obuf[j, csl] = wa * ya[j, csl] + wb * yb[j, csl]
            return 0

        lax.fori_loop(0, 16, jbody, 0)
        oh[c] = pltpu.async_copy(obuf, out_hbm.at[pl.ds(base + c * 16, 16)],
                                 osem)
        if c + 2 < nch:
            handles[c + 2] = start(c + 2)
    oh[nch - 1].wait()


def _sc_mesh():
    return plsc.VectorSubcoreMesh(core_axis_name="c", subcore_axis_name="s")


@jax.jit
def kernel(hidden_states, router_w, w1, w2, shared_w1, shared_w2):
    rwp = jnp.pad(router_w, ((0, 0), (0, EP - E)))

    meta_t, counts, aux = pl.pallas_call(
        _router_body,
        grid=(T // RTB,),
        in_specs=[
            pl.BlockSpec((RTB, H), lambda i: (i, 0)),
            pl.BlockSpec((H, EP), lambda i: (0, 0)),
        ],
        out_specs=[
            pl.BlockSpec((EP, RTB), lambda i: (0, i)),
            pl.BlockSpec((1, EP), lambda i: (0, 0)),
            pl.BlockSpec((1, 1), lambda i: (0, 0)),
        ],
        out_shape=[
            jax.ShapeDtypeStruct((EP, T), jnp.float32),
            jax.ShapeDtypeStruct((1, EP), jnp.float32),
            jax.ShapeDtypeStruct((1, 1), jnp.float32),
        ],
        scratch_shapes=[
            pltpu.VMEM((1, EP), jnp.float32),
            pltpu.VMEM((1, EP), jnp.float32),
        ],
    )(hidden_states, rwp)

    dispatch = functools.partial(
        pl.kernel,
        mesh=_sc_mesh(),
        out_type=[
            jax.ShapeDtypeStruct((NROWS, H), jnp.float32),
            jax.ShapeDtypeStruct((16,), jnp.int32),
            jax.ShapeDtypeStruct((NBLK_PAD,), jnp.int32),
        ],
        scratch_types=[
            pltpu.VMEM((TPW, H), jnp.float32),
            pltpu.VMEM((8, TPW), jnp.float32),
            pltpu.VMEM((16,), jnp.float32),
            pltpu.VMEM((2, TPW), jnp.int32),
            pltpu.VMEM((16,), jnp.int32),
            pltpu.VMEM((NBLK_PAD,), jnp.int32),
            pltpu.SemaphoreType.DMA,
        ],
    )
    xs, cnt16, bexp = dispatch(_dispatch_body)(hidden_states, meta_t, counts)

    ys = pl.pallas_call(
        _group_mlp_body,
        grid_spec=pltpu.PrefetchScalarGridSpec(
            num_scalar_prefetch=2,
            grid=(NBLK,),
            in_specs=[
                pl.BlockSpec((BLK, H), lambda s, cnt, bexp: (s, 0)),
                pl.BlockSpec((1, H, F), lambda s, cnt, bexp: (bexp[s], 0, 0)),
                pl.BlockSpec((1, F, H), lambda s, cnt, bexp: (bexp[s], 0, 0)),
            ],
            out_specs=pl.BlockSpec((BLK, H), lambda s, cnt, bexp: (s, 0)),
        ),
        out_shape=jax.ShapeDtypeStruct((NROWS, H), jnp.float32),
        compiler_params=pltpu.CompilerParams(
            dimension_semantics=("arbitrary",)),
    )(cnt16, bexp, xs, w1, w2)

    combine = functools.partial(
        pl.kernel,
        mesh=_sc_mesh(),
        out_type=jax.ShapeDtypeStruct((T, H), jnp.float32),
        scratch_types=[
            pltpu.VMEM((16, H), jnp.float32),
            pltpu.VMEM((16, H), jnp.float32),
            pltpu.VMEM((16, H), jnp.float32),
            pltpu.VMEM((16, H), jnp.float32),
            pltpu.VMEM((16, H), jnp.float32),
            pltpu.VMEM((8, TPW), jnp.float32),
            pltpu.VMEM((16,), jnp.float32),
            pltpu.VMEM((4, 16), jnp.int32),
            pltpu.SemaphoreType.DMA,
            pltpu.SemaphoreType.DMA,
            pltpu.SemaphoreType.DMA,
        ],
    )
    moe = combine(_combine_body)(ys, meta_t, counts)

    combined = pl.pallas_call(
        _shared_body,
        grid=(T // XTB,),
        in_specs=[
            pl.BlockSpec((XTB, H), lambda t: (t, 0)),
            pl.BlockSpec((H, FS), lambda t: (0, 0)),
            pl.BlockSpec((FS, H), lambda t: (0, 0)),
            pl.BlockSpec((XTB, H), lambda t: (t, 0)),
        ],
        out_specs=pl.BlockSpec((XTB, H), lambda t: (t, 0)),
        out_shape=jax.ShapeDtypeStruct((T, H), jnp.float32),
    )(hidden_states, shared_w1, shared_w2, moe)

    return combined, aux[0, 0]
